# baseline (speedup 1.0000x reference)
import functools

import jax
import jax.numpy as jnp
from jax import lax
from jax.experimental import pallas as pl
from jax.experimental.pallas import tpu as pltpu
from jax.experimental.pallas import tpu_sc as plsc

NUM_NUM = 13
NUM_CAT = 26
VOCAB = 100000
D = 64
B = 4096
TOK = NUM_NUM + NUM_CAT
L = 16

_INFO = plsc.get_sparse_core_info()
NC = _INFO.num_cores
NS = _INFO.num_subcores
NW = NC * NS
ROWS = B // NW
S = 8
HS = 4
CHUNKS = ROWS // S
HALVES = ROWS // HS


def _tokenizer_body(x_hbm, emb_hbm, tab_hbm, out_hbm,
                    xv, embv, pidx,
                    stg0, stg1, slot0, slot1,
                    gsem0, gsem1, wsem0, wsem1):
    stgs = (stg0, stg1)
    slots = (slot0, slot1)
    gsems = (gsem0, gsem1)
    wsems = (wsem0, wsem1)

    wid = lax.axis_index("s") * NC + lax.axis_index("c")
    base = wid * ROWS

    pltpu.sync_copy(x_hbm.at[pl.ds(base, ROWS)], xv)
    pltpu.sync_copy(emb_hbm, embv)

    def idx_step(b, carry):
        bb = jnp.full((L,), b, jnp.int32)
        row = lax.shift_right_logical(bb, 2)
        lane0 = jnp.bitwise_and(bb, 3) * NUM_CAT
        for cbase in (0, NUM_CAT - L):
            cc = lax.iota(jnp.int32, L) + cbase
            vals = plsc.load_gather(xv, [bb, cc + NUM_NUM])
            gidx = vals.astype(jnp.int32) + cc * VOCAB
            plsc.store_scatter(pidx, [row, lane0 + cc],
                               lax.shift_right_logical(gidx, 1))
        return carry

    lax.fori_loop(0, ROWS, idx_step, 0)

    def fire_half(b0, h):
        stg, gsem = stgs[h % 2], gsems[h % 2]
        pltpu.async_copy(tab_hbm.at[pidx.at[b0 // HS]], stg, gsem)

    def wait_half(b0, h):
        stg, gsem = stgs[h % 2], gsems[h % 2]
        pltpu.make_async_copy(
            tab_hbm.at[pidx.at[b0 // HS]], stg, gsem
        ).wait()

    def write_tile(t, p):
        slot, wsem = slots[p], wsems[p]
        pltpu.async_copy(
            slot, out_hbm.at[pl.ds((base + t * S) * TOK, S * TOK)], wsem
        )

    def wait_write(t, p):
        slot, wsem = slots[p], wsems[p]
        pltpu.make_async_copy(
            slot, out_hbm.at[pl.ds((base + t * S) * TOK, S * TOK)], wsem
        ).wait()

    def compute_half(b0, h, p):
        stg, slot = stgs[h % 2], slots[p]
        r0 = h * HS

        def num_step(rc, carry):
            r = rc // NUM_NUM
            c = rc % NUM_NUM
            bb = jnp.full((L,), b0 + r, jnp.int32)
            cc = jnp.full((L,), c, jnp.int32)
            sval = plsc.load_gather(xv, [bb, cc])
            row = jnp.full((L,), (r0 + r) * TOK + c, jnp.int32)
            for dd in range(D // L):
                ev = plsc.load_gather(
                    embv, [cc, lax.iota(jnp.int32, L) + dd * L]
                )
                plsc.store_scatter(
                    slot, [row, lax.iota(jnp.int32, L) + dd * L], sval * ev
                )
            return carry

        lax.fori_loop(0, HS * NUM_NUM, num_step, 0)

        def sel_step(j, carry):
            r = j // NUM_CAT
            c = j % NUM_CAT
            jj = jnp.full((L,), j, jnp.int32)
            xval = plsc.load_gather(
                xv,
                [jnp.full((L,), b0, jnp.int32) + r,
                 jnp.full((L,), NUM_NUM, jnp.int32) + c],
            )
            paroff = lax.shift_left(
                jnp.bitwise_and(xval.astype(jnp.int32), 1), 6
            )
            row = jnp.full((L,), (r0 + r) * TOK + NUM_NUM + c, jnp.int32)
            for dd in range(D // L):
                lanes = lax.iota(jnp.int32, L) + dd * L
                v = plsc.load_gather(stg, [jj, lanes + paroff])
                plsc.store_scatter(slot, [row, lanes], v)
            return carry

        lax.fori_loop(0, HS * NUM_CAT, sel_step, 0)

    Q = S // HS

    for h in range(Q):
        fire_half(h * HS, h)

    def tile(t, p):
        b0 = t * S

        @pl.when(t >= 2)
        def _():
            wait_write(t - 2, p)

        for h in range(Q):
            wait_half(b0 + h * HS, h)
            compute_half(b0 + h * HS, h, p)

            @pl.when(t + 1 < CHUNKS)
            def _(h=h):
                fire_half(b0 + S + h * HS, h)
        write_tile(t, p)

    def tile_pair(tt, carry):
        tile(2 * tt, 0)
        tile(2 * tt + 1, 1)
        return carry

    lax.fori_loop(0, CHUNKS // 2, tile_pair, 0)
    wait_write(CHUNKS - 2, 0)
    wait_write(CHUNKS - 1, 1)


_tokenizer = functools.partial(
    pl.kernel,
    out_type=jax.ShapeDtypeStruct((B * TOK, D), jnp.float32),
    mesh=plsc.VectorSubcoreMesh(core_axis_name="c", subcore_axis_name="s"),
    compiler_params=pltpu.CompilerParams(needs_layout_passes=False),
    scratch_types=[
        pltpu.VMEM((ROWS, TOK), jnp.float32),
        pltpu.VMEM((NUM_NUM, D), jnp.float32),
        pltpu.VMEM((ROWS // HS, HS * NUM_CAT), jnp.int32),
        pltpu.VMEM((HS * NUM_CAT, 2 * D), jnp.float32),
        pltpu.VMEM((HS * NUM_CAT, 2 * D), jnp.float32),
        pltpu.VMEM((S * TOK, D), jnp.float32),
        pltpu.VMEM((S * TOK, D), jnp.float32),
        pltpu.SemaphoreType.DMA,
        pltpu.SemaphoreType.DMA,
        pltpu.SemaphoreType.DMA,
        pltpu.SemaphoreType.DMA,
    ],
)(_tokenizer_body)


def kernel(x, numerical_embeddings, categorical_tables):
    pair_table = jnp.concatenate(
        [categorical_tables[:, 0::2, :], categorical_tables[:, 1::2, :]],
        axis=2,
    ).reshape(NUM_CAT * VOCAB // 2, 2 * D)
    out = _tokenizer(x, numerical_embeddings, pair_table)
    return out.reshape(B, TOK, D)


# speedup vs baseline: 14.2114x; 14.2114x over previous
import functools

import jax
import jax.numpy as jnp
from jax import lax
from jax.experimental import pallas as pl
from jax.experimental.pallas import tpu as pltpu
from jax.experimental.pallas import tpu_sc as plsc

NUM_NUM = 13
NUM_CAT = 26
VOCAB = 100000
D = 64
B = 4096
TOK = NUM_NUM + NUM_CAT
L = 16

_INFO = plsc.get_sparse_core_info()
NC = _INFO.num_cores
NS = _INFO.num_subcores
NW = NC * NS
ROWS = B // NW
S = 8
CHUNKS = ROWS // S


def _tokenizer_body(x_hbm, emb_hbm, tab_hbm, out_hbm,
                    xv, embv, idxv, slot0, slot1,
                    gsem0, gsem1, wsem0, wsem1):
    slots = (slot0, slot1)
    gsems = (gsem0, gsem1)
    wsems = (wsem0, wsem1)

    wid = lax.axis_index("s") * NC + lax.axis_index("c")
    base = wid * ROWS

    pltpu.sync_copy(x_hbm.at[pl.ds(base, ROWS)], xv)
    pltpu.sync_copy(emb_hbm, embv)

    def idx_step(b, carry):
        bb = jnp.full((L,), b, jnp.int32)
        for cbase in (0, NUM_CAT - L):
            cc = lax.iota(jnp.int32, L) + cbase
            vals = plsc.load_gather(xv, [bb, cc + NUM_NUM])
            plsc.store_scatter(idxv, [bb, cc],
                               vals.astype(jnp.int32) + cc * VOCAB)
        return carry

    lax.fori_loop(0, ROWS, idx_step, 0)

    def fire_gathers(t, p):
        slot, gsem = slots[p], gsems[p]
        for r in range(S):
            pltpu.async_copy(
                tab_hbm.at[idxv.at[t * S + r]],
                slot.at[pl.ds(r * TOK + NUM_NUM, NUM_CAT)],
                gsem,
            )

    def wait_gathers(t, p):
        slot, gsem = slots[p], gsems[p]
        for r in range(S):
            pltpu.make_async_copy(
                tab_hbm.at[idxv.at[t * S + r]],
                slot.at[pl.ds(r * TOK + NUM_NUM, NUM_CAT)],
                gsem,
            ).wait()

    def write_tile(t, p):
        slot, wsem = slots[p], wsems[p]
        pltpu.async_copy(
            slot, out_hbm.at[pl.ds((base + t * S) * TOK, S * TOK)], wsem
        )

    def wait_write(t, p):
        slot, wsem = slots[p], wsems[p]
        pltpu.make_async_copy(
            slot, out_hbm.at[pl.ds((base + t * S) * TOK, S * TOK)], wsem
        ).wait()

    def compute_num(t, p):
        slot = slots[p]
        b0 = t * S

        def num_step(rc, carry):
            r = rc // NUM_NUM
            c = rc % NUM_NUM
            bb = jnp.full((L,), b0 + r, jnp.int32)
            cc = jnp.full((L,), c, jnp.int32)
            sval = plsc.load_gather(xv, [bb, cc])
            row = jnp.full((L,), r * TOK + c, jnp.int32)
            for dd in range(D // L):
                ev = plsc.load_gather(
                    embv, [cc, lax.iota(jnp.int32, L) + dd * L]
                )
                plsc.store_scatter(
                    slot, [row, lax.iota(jnp.int32, L) + dd * L], sval * ev
                )
            return carry

        lax.fori_loop(0, S * NUM_NUM, num_step, 0)

    def tile(t, p):
        @pl.when(jnp.logical_and(t >= 1, t + 1 < CHUNKS))
        def _():
            wait_write(t - 1, 1 - p)

        @pl.when(t + 1 < CHUNKS)
        def _():
            fire_gathers(t + 1, 1 - p)

        compute_num(t, p)
        wait_gathers(t, p)
        write_tile(t, p)

    def tile_pair(tt, carry):
        tile(2 * tt, 0)
        tile(2 * tt + 1, 1)
        return carry

    fire_gathers(0, 0)
    lax.fori_loop(0, CHUNKS // 2, tile_pair, 0)
    wait_write(CHUNKS - 2, 0)
    wait_write(CHUNKS - 1, 1)


_tokenizer = functools.partial(
    pl.kernel,
    out_type=jax.ShapeDtypeStruct((B * TOK, D), jnp.float32),
    mesh=plsc.VectorSubcoreMesh(core_axis_name="c", subcore_axis_name="s"),
    compiler_params=pltpu.CompilerParams(
        use_tc_tiling_on_sc=False, needs_layout_passes=False
    ),
    scratch_types=[
        pltpu.VMEM((ROWS, TOK), jnp.float32),
        pltpu.VMEM((NUM_NUM, D), jnp.float32),
        pltpu.VMEM((ROWS, NUM_CAT), jnp.int32),
        pltpu.VMEM((S * TOK, D), jnp.float32),
        pltpu.VMEM((S * TOK, D), jnp.float32),
        pltpu.SemaphoreType.DMA,
        pltpu.SemaphoreType.DMA,
        pltpu.SemaphoreType.DMA,
        pltpu.SemaphoreType.DMA,
    ],
)(_tokenizer_body)


def kernel(x, numerical_embeddings, categorical_tables):
    flat_table = categorical_tables.reshape(NUM_CAT * VOCAB, D)
    out = _tokenizer(x, numerical_embeddings, flat_table)
    return out.reshape(B, TOK, D)


# speedup vs baseline: 24.9459x; 1.7553x over previous
import functools

import jax
import jax.numpy as jnp
from jax import lax
from jax.experimental import pallas as pl
from jax.experimental.pallas import tpu as pltpu
from jax.experimental.pallas import tpu_sc as plsc

NUM_NUM = 13
NUM_CAT = 26
VOCAB = 100000
D = 64
B = 4096
TOK = NUM_NUM + NUM_CAT
L = 16

_INFO = plsc.get_sparse_core_info()
NC = _INFO.num_cores
NS = _INFO.num_subcores
NW = NC * NS
ROWS = B // NW
S = 8
CHUNKS = ROWS // S
G = 8
NGRP = S * NUM_CAT // G


def _tokenizer_body(x_hbm, emb_hbm, tab_hbm, out_hbm,
                    xv, embv, idxv, stg, slot0, slot1,
                    gsem0, gsem1, wsem0, wsem1):
    slots = (slot0, slot1)
    gsems = (gsem0, gsem1)
    wsems = (wsem0, wsem1)

    wid = lax.axis_index("s") * NC + lax.axis_index("c")
    base = wid * ROWS

    pltpu.sync_copy(x_hbm.at[pl.ds(base, ROWS)], xv)
    pltpu.sync_copy(emb_hbm, embv)

    def idx_step(b, carry):
        bb = jnp.full((L,), b, jnp.int32)
        for cbase in (0, NUM_CAT - L):
            cc = lax.iota(jnp.int32, L) + cbase
            vals = plsc.load_gather(xv, [bb, cc + NUM_NUM])
            plsc.store_scatter(idxv, [bb, cc],
                               vals.astype(jnp.int32) + cc * VOCAB)
        return carry

    lax.fori_loop(0, ROWS, idx_step, 0)

    def fire_group(t, g, gp):
        def fire_one(k, carry):
            j = g * G + k
            gvec = plsc.load_gather(
                idxv,
                [jnp.full((L,), t * S, jnp.int32) + j // NUM_CAT,
                 jnp.full((L,), j % NUM_CAT, jnp.int32)],
            )
            gidx = jnp.max(gvec)
            blk = pl.multiple_of((gidx // S) * S, S)
            pltpu.async_copy(
                tab_hbm.at[pl.ds(blk, S)],
                stg.at[pl.ds((gp * G + k) * S, S)],
                gsems[gp],
            )
            return carry

        lax.fori_loop(0, G, fire_one, 0)

    def wait_group(gp):
        def wait_one(k, carry):
            pltpu.make_async_copy(
                tab_hbm.at[pl.ds(0, S)],
                stg.at[pl.ds((gp * G + k) * S, S)],
                gsems[gp],
            ).wait()
            return carry

        lax.fori_loop(0, G, wait_one, 0)

    def write_tile(t, p):
        slot, wsem = slots[p], wsems[p]
        pltpu.async_copy(
            slot, out_hbm.at[pl.ds((base + t * S) * TOK, S * TOK)], wsem
        )

    def wait_write(t, p):
        slot, wsem = slots[p], wsems[p]
        pltpu.make_async_copy(
            slot, out_hbm.at[pl.ds((base + t * S) * TOK, S * TOK)], wsem
        ).wait()

    def compute_num(t, p):
        slot = slots[p]
        b0 = t * S

        def num_step(rc, carry):
            r = rc // NUM_NUM
            c = rc % NUM_NUM
            bb = jnp.full((L,), b0 + r, jnp.int32)
            cc = jnp.full((L,), c, jnp.int32)
            sval = plsc.load_gather(xv, [bb, cc])
            row = jnp.full((L,), r * TOK + c, jnp.int32)
            for dd in range(D // L):
                ev = plsc.load_gather(
                    embv, [cc, lax.iota(jnp.int32, L) + dd * L]
                )
                plsc.store_scatter(
                    slot, [row, lax.iota(jnp.int32, L) + dd * L], sval * ev
                )
            return carry

        lax.fori_loop(0, S * NUM_NUM, num_step, 0)

    def lookups(t, p):
        slot = slots[p]

        def half(g, gp):
            wait_group(gp)

            def copy_one(k, carry):
                j = g * G + k
                gvec = plsc.load_gather(
                    idxv,
                    [jnp.full((L,), t * S, jnp.int32) + j // NUM_CAT,
                     jnp.full((L,), j % NUM_CAT, jnp.int32)],
                )
                off = jnp.full((L,), (gp * G + k) * S, jnp.int32) + gvec % S
                row = (jnp.full((L,), NUM_NUM, jnp.int32)
                       + (j // NUM_CAT) * TOK + j % NUM_CAT)
                for dd in range(D // L):
                    lanes = lax.iota(jnp.int32, L) + dd * L
                    v = plsc.load_gather(stg, [off, lanes])
                    plsc.store_scatter(slot, [row, lanes], v)
                return carry

            lax.fori_loop(0, G, copy_one, 0)

            @pl.when(g + 2 < NGRP)
            def _():
                fire_group(t, g + 2, gp)

        def body(gg, carry):
            half(2 * gg, 0)
            half(2 * gg + 1, 1)
            return carry

        fire_group(t, 0, 0)
        fire_group(t, 1, 1)
        lax.fori_loop(0, NGRP // 2, body, 0)

    def tile(t, p):
        @pl.when(t >= 2)
        def _():
            wait_write(t - 2, p)

        compute_num(t, p)
        lookups(t, p)
        write_tile(t, p)

    def tile_pair(tt, carry):
        tile(2 * tt, 0)
        tile(2 * tt + 1, 1)
        return carry

    lax.fori_loop(0, CHUNKS // 2, tile_pair, 0)
    wait_write(CHUNKS - 2, 0)
    wait_write(CHUNKS - 1, 1)


_tokenizer = functools.partial(
    pl.kernel,
    out_type=jax.ShapeDtypeStruct((B * TOK, D), jnp.float32),
    mesh=plsc.VectorSubcoreMesh(core_axis_name="c", subcore_axis_name="s"),
    compiler_params=pltpu.CompilerParams(needs_layout_passes=False),
    scratch_types=[
        pltpu.VMEM((ROWS, TOK), jnp.float32),
        pltpu.VMEM((NUM_NUM, D), jnp.float32),
        pltpu.VMEM((ROWS, NUM_CAT), jnp.int32),
        pltpu.VMEM((2 * G * S, D), jnp.float32),
        pltpu.VMEM((S * TOK, D), jnp.float32),
        pltpu.VMEM((S * TOK, D), jnp.float32),
        pltpu.SemaphoreType.DMA,
        pltpu.SemaphoreType.DMA,
        pltpu.SemaphoreType.DMA,
        pltpu.SemaphoreType.DMA,
    ],
)(_tokenizer_body)


def kernel(x, numerical_embeddings, categorical_tables):
    flat_table = categorical_tables.reshape(NUM_CAT * VOCAB, D)
    return _tokenizer(x, numerical_embeddings, flat_table).reshape(B, TOK, D)


# speedup vs baseline: 26.0220x; 1.0431x over previous
import functools

import jax
import jax.numpy as jnp
from jax import lax
from jax.experimental import pallas as pl
from jax.experimental.pallas import tpu as pltpu
from jax.experimental.pallas import tpu_sc as plsc

NUM_NUM = 13
NUM_CAT = 26
VOCAB = 100000
D = 64
B = 4096
TOK = NUM_NUM + NUM_CAT
L = 16

_INFO = plsc.get_sparse_core_info()
NC = _INFO.num_cores
NS = _INFO.num_subcores
NW = NC * NS
ROWS = B // NW
S = 8
CHUNKS = ROWS // S
G = 4
NGRP = S * NUM_CAT // G
RING = 4


def _tokenizer_body(x_hbm, emb_hbm, tab_hbm, out_hbm,
                    xv, embv, idxv, stg, slot0, slot1,
                    gsem0, gsem1, gsem2, gsem3, wsem0, wsem1):
    slots = (slot0, slot1)
    gsems = (gsem0, gsem1, gsem2, gsem3)
    wsems = (wsem0, wsem1)

    wid = lax.axis_index("s") * NC + lax.axis_index("c")
    base = wid * ROWS

    pltpu.sync_copy(x_hbm.at[pl.ds(base, ROWS)], xv)
    pltpu.sync_copy(emb_hbm, embv)

    def idx_step(b, carry):
        bb = jnp.full((L,), b, jnp.int32)
        for cbase in (0, NUM_CAT - L):
            cc = lax.iota(jnp.int32, L) + cbase
            vals = plsc.load_gather(xv, [bb, cc + NUM_NUM])
            plsc.store_scatter(idxv, [bb, cc],
                               vals.astype(jnp.int32) + cc * VOCAB)
        return carry

    lax.fori_loop(0, ROWS, idx_step, 0)

    def fire_group(t, g, gp):
        def fire_one(k, carry):
            j = g * G + k
            gvec = plsc.load_gather(
                idxv,
                [jnp.full((L,), t * S, jnp.int32) + j // NUM_CAT,
                 jnp.full((L,), j % NUM_CAT, jnp.int32)],
            )
            gidx = jnp.max(gvec)
            blk = pl.multiple_of((gidx // S) * S, S)
            pltpu.async_copy(
                tab_hbm.at[pl.ds(blk, S)],
                stg.at[pl.ds((gp * G + k) * S, S)],
                gsems[gp],
            )
            return carry

        lax.fori_loop(0, G, fire_one, 0)

    def wait_group(gp):
        def wait_one(k, carry):
            pltpu.make_async_copy(
                tab_hbm.at[pl.ds(0, S)],
                stg.at[pl.ds((gp * G + k) * S, S)],
                gsems[gp],
            ).wait()
            return carry

        lax.fori_loop(0, G, wait_one, 0)

    def write_tile(t, p):
        slot, wsem = slots[p], wsems[p]
        pltpu.async_copy(
            slot, out_hbm.at[pl.ds((base + t * S) * TOK, S * TOK)], wsem
        )

    def wait_write(t, p):
        slot, wsem = slots[p], wsems[p]
        pltpu.make_async_copy(
            slot, out_hbm.at[pl.ds((base + t * S) * TOK, S * TOK)], wsem
        ).wait()

    def compute_num(t, p):
        slot = slots[p]
        b0 = t * S

        def num_step(rc, carry):
            r = rc // NUM_NUM
            c = rc % NUM_NUM
            bb = jnp.full((L,), b0 + r, jnp.int32)
            cc = jnp.full((L,), c, jnp.int32)
            sval = plsc.load_gather(xv, [bb, cc])
            row = jnp.full((L,), r * TOK + c, jnp.int32)
            for dd in range(D // L):
                ev = plsc.load_gather(
                    embv, [cc, lax.iota(jnp.int32, L) + dd * L]
                )
                plsc.store_scatter(
                    slot, [row, lax.iota(jnp.int32, L) + dd * L], sval * ev
                )
            return carry

        lax.fori_loop(0, S * NUM_NUM, num_step, 0)

    def lookups(t, p):
        slot = slots[p]

        def half(g, gp):
            wait_group(gp)

            def copy_one(k, carry):
                j = g * G + k
                gvec = plsc.load_gather(
                    idxv,
                    [jnp.full((L,), t * S, jnp.int32) + j // NUM_CAT,
                     jnp.full((L,), j % NUM_CAT, jnp.int32)],
                )
                off = jnp.full((L,), (gp * G + k) * S, jnp.int32) + gvec % S
                row = (jnp.full((L,), NUM_NUM, jnp.int32)
                       + (j // NUM_CAT) * TOK + j % NUM_CAT)
                for dd in range(D // L):
                    lanes = lax.iota(jnp.int32, L) + dd * L
                    v = plsc.load_gather(stg, [off, lanes])
                    plsc.store_scatter(slot, [row, lanes], v)
                return carry

            lax.fori_loop(0, G, copy_one, 0)

            @pl.when(g + RING < NGRP)
            def _():
                fire_group(t, g + RING, gp)

        def body(gg, carry):
            for q in range(RING):
                half(RING * gg + q, q)
            return carry

        for q in range(RING):
            fire_group(t, q, q)
        lax.fori_loop(0, NGRP // RING, body, 0)
        for g in range(NGRP - NGRP % RING, NGRP):
            half(jnp.int32(g), g % RING)

    def tile(t, p):
        @pl.when(t >= 2)
        def _():
            wait_write(t - 2, p)

        compute_num(t, p)
        lookups(t, p)
        write_tile(t, p)

    def tile_pair(tt, carry):
        tile(2 * tt, 0)
        tile(2 * tt + 1, 1)
        return carry

    lax.fori_loop(0, CHUNKS // 2, tile_pair, 0)
    wait_write(CHUNKS - 2, 0)
    wait_write(CHUNKS - 1, 1)


_tokenizer = functools.partial(
    pl.kernel,
    out_type=jax.ShapeDtypeStruct((B * TOK, D), jnp.float32),
    mesh=plsc.VectorSubcoreMesh(core_axis_name="c", subcore_axis_name="s"),
    compiler_params=pltpu.CompilerParams(needs_layout_passes=False),
    scratch_types=[
        pltpu.VMEM((ROWS, TOK), jnp.float32),
        pltpu.VMEM((NUM_NUM, D), jnp.float32),
        pltpu.VMEM((ROWS, NUM_CAT), jnp.int32),
        pltpu.VMEM((RING * G * S, D), jnp.float32),
        pltpu.VMEM((S * TOK, D), jnp.float32),
        pltpu.VMEM((S * TOK, D), jnp.float32),
        pltpu.SemaphoreType.DMA,
        pltpu.SemaphoreType.DMA,
        pltpu.SemaphoreType.DMA,
        pltpu.SemaphoreType.DMA,
        pltpu.SemaphoreType.DMA,
        pltpu.SemaphoreType.DMA,
    ],
)(_tokenizer_body)


def kernel(x, numerical_embeddings, categorical_tables):
    flat_table = categorical_tables.reshape(NUM_CAT * VOCAB, D)
    return _tokenizer(x, numerical_embeddings, flat_table).reshape(B, TOK, D)
